# baseline (speedup 1.0000x reference)
import jax
import jax.numpy as jnp
from jax.experimental import pallas as pl
from jax.experimental.pallas import tpu as pltpu

_SIZES = [32, 32, 64, 128, 256] + [1024] * 7 + [256, 128, 64, 32, 32]
_POOL = 3584
_LOOKAHEAD = 3


def _plan():
    hbm_off, pool_off = [], []
    h = 0
    c = 0
    for sz in _SIZES:
        if c + sz > _POOL:
            c = 0
        hbm_off.append(h)
        pool_off.append(c)
        h += sz
        c += sz
    assert h == 8192
    return hbm_off, pool_off


def _ring_body(emb_hbm, out_hbm, pool, in_sems, out_sems):
    nch = len(_SIZES)
    hbm_off, pool_off = _plan()

    def in_copy(i):
        return pltpu.make_async_copy(
            emb_hbm.at[pl.ds(hbm_off[i], _SIZES[i])],
            pool.at[pl.ds(pool_off[i], _SIZES[i])],
            in_sems.at[i],
        )

    def out_copy(i):
        return pltpu.make_async_copy(
            pool.at[pl.ds(pool_off[i], _SIZES[i])],
            out_hbm.at[pl.ds(hbm_off[i], _SIZES[i])],
            out_sems.at[i],
        )

    waited = set()

    def start_load(j):
        lo, hi = pool_off[j], pool_off[j] + _SIZES[j]
        for k in range(j):
            if k in waited:
                continue
            klo, khi = pool_off[k], pool_off[k] + _SIZES[k]
            if klo < hi and lo < khi:
                out_copy(k).wait()
                waited.add(k)
        in_copy(j).start()

    for j in range(min(_LOOKAHEAD, nch)):
        start_load(j)
    for i in range(nch):
        in_copy(i).wait()
        out_copy(i).start()
        j = i + _LOOKAHEAD
        if j < nch:
            start_load(j)
    for k in range(nch):
        if k not in waited:
            out_copy(k).wait()


def kernel(emb, t):
    del t
    n, d = emb.shape
    nch = len(_SIZES)
    return pl.pallas_call(
        _ring_body,
        in_specs=[pl.BlockSpec(memory_space=pl.ANY)],
        out_specs=pl.BlockSpec(memory_space=pl.ANY),
        out_shape=jax.ShapeDtypeStruct((n, d), emb.dtype),
        scratch_shapes=[
            pltpu.VMEM((_POOL, d), jnp.float32),
            pltpu.SemaphoreType.DMA((nch,)),
            pltpu.SemaphoreType.DMA((nch,)),
        ],
        compiler_params=pltpu.CompilerParams(skip_device_barrier=True),
    )(emb)


# speedup vs baseline: 1.0098x; 1.0098x over previous
import jax
import jax.numpy as jnp
from jax.experimental import pallas as pl
from jax.experimental.pallas import tpu as pltpu

_SIZES = [32, 32, 64, 128, 256] + [512] * 14 + [256, 128, 64, 32, 32]
_POOL = 3840
_LOOKAHEAD = 7


def _plan():
    hbm_off, pool_off = [], []
    h = 0
    c = 0
    for sz in _SIZES:
        if c + sz > _POOL:
            c = 0
        hbm_off.append(h)
        pool_off.append(c)
        h += sz
        c += sz
    assert h == 8192
    return hbm_off, pool_off


def _ring_body(emb_hbm, out_hbm, pool, in_sems, out_sems):
    nch = len(_SIZES)
    hbm_off, pool_off = _plan()

    def in_copy(i):
        return pltpu.make_async_copy(
            emb_hbm.at[pl.ds(hbm_off[i], _SIZES[i])],
            pool.at[pl.ds(pool_off[i], _SIZES[i])],
            in_sems.at[i],
        )

    def out_copy(i):
        return pltpu.make_async_copy(
            pool.at[pl.ds(pool_off[i], _SIZES[i])],
            out_hbm.at[pl.ds(hbm_off[i], _SIZES[i])],
            out_sems.at[i],
        )

    waited = set()

    def start_load(j):
        lo, hi = pool_off[j], pool_off[j] + _SIZES[j]
        for k in range(j):
            if k in waited:
                continue
            klo, khi = pool_off[k], pool_off[k] + _SIZES[k]
            if klo < hi and lo < khi:
                out_copy(k).wait()
                waited.add(k)
        in_copy(j).start()

    for j in range(min(_LOOKAHEAD, nch)):
        start_load(j)
    for i in range(nch):
        in_copy(i).wait()
        out_copy(i).start()
        j = i + _LOOKAHEAD
        if j < nch:
            start_load(j)
    for k in range(nch):
        if k not in waited:
            out_copy(k).wait()


def kernel(emb, t):
    del t
    n, d = emb.shape
    nch = len(_SIZES)
    return pl.pallas_call(
        _ring_body,
        in_specs=[pl.BlockSpec(memory_space=pl.ANY)],
        out_specs=pl.BlockSpec(memory_space=pl.ANY),
        out_shape=jax.ShapeDtypeStruct((n, d), emb.dtype),
        scratch_shapes=[
            pltpu.VMEM((_POOL, d), jnp.float32),
            pltpu.SemaphoreType.DMA((nch,)),
            pltpu.SemaphoreType.DMA((nch,)),
        ],
        compiler_params=pltpu.CompilerParams(skip_device_barrier=True, vmem_limit_bytes=63 * 1024 * 1024),
    )(emb)
